# baseline (speedup 1.0000x reference)
import functools

import jax
import jax.numpy as jnp
from jax.experimental import pallas as pl
from jax.experimental.pallas import tpu as pltpu
from jax.experimental.pallas import tpu_sc as plsc

VOCAB = 20000
EMB = 50
SEQ = 42
BATCH = 16384
HID = 128
DPAD = 128
SPAD = 48
ROWS_PER_STEP = 8
WINDOW = ROWS_PER_STEP * SPAD
BLOCK_B = 512


def _sc_gather(table_pad, idx2d):
    mesh = plsc.VectorSubcoreMesh(core_axis_name="core", subcore_axis_name="subcore")

    @functools.partial(
        pl.kernel,
        out_type=jax.ShapeDtypeStruct((BATCH, SPAD, DPAD), table_pad.dtype),
        mesh=mesh,
    )
    def gather_kernel(table_hbm, i_hbm, o_hbm):
        def body(i_vmem, o_vmem):
            ov = o_vmem.reshape(WINDOW, DPAD)
            for k in range(WINDOW // 128):
                pltpu.sync_copy(
                    table_hbm.at[i_vmem.at[0, pl.ds(128 * k, 128)]],
                    ov.at[pl.ds(128 * k, 128)],
                )

        pltpu.emit_pipeline(
            body,
            grid=(BATCH // ROWS_PER_STEP,),
            in_specs=[pl.BlockSpec((1, WINDOW), lambda i: (i, 0))],
            out_specs=[
                pl.BlockSpec((ROWS_PER_STEP, SPAD, DPAD), lambda i: (i, 0, 0))
            ],
            core_axis_name=("core", "subcore"),
            dimension_semantics=(pltpu.PARALLEL,),
        )(i_hbm, o_hbm)

    return gather_kernel(table_pad, idx2d)


def _mlp_body(x_ref, w1_ref, b1_ref, w2_ref, b2_ref, o_ref):
    acc = jnp.dot(x_ref[:, 0, :], w1_ref[0], preferred_element_type=jnp.float32)
    for s in range(1, SEQ):
        acc += jnp.dot(
            x_ref[:, s, :], w1_ref[s], preferred_element_type=jnp.float32
        )
    h = jnp.maximum(acc + b1_ref[...], 0.0)
    o = jnp.dot(h, w2_ref[...], preferred_element_type=jnp.float32) + b2_ref[...]
    o_ref[...] = jax.nn.sigmoid(o)


def _tc_mlp(x3, w1r, b1, w2, b2):
    grid = (BATCH // BLOCK_B,)
    return pl.pallas_call(
        _mlp_body,
        grid=grid,
        in_specs=[
            pl.BlockSpec((BLOCK_B, SPAD, DPAD), lambda i: (i, 0, 0)),
            pl.BlockSpec((SEQ, DPAD, HID), lambda i: (0, 0, 0)),
            pl.BlockSpec((1, HID), lambda i: (0, 0)),
            pl.BlockSpec((HID, 1), lambda i: (0, 0)),
            pl.BlockSpec((1, 1), lambda i: (0, 0)),
        ],
        out_specs=pl.BlockSpec((BLOCK_B, 1), lambda i: (i, 0)),
        out_shape=jax.ShapeDtypeStruct((BATCH, 1), jnp.float32),
    )(x3, w1r, b1.reshape(1, HID), w2, b2.reshape(1, 1))


def kernel(indices, table, W1, b1, W2, b2):
    table_pad = jnp.pad(table, ((0, 0), (0, DPAD - EMB)))
    idxp = jnp.pad(indices.astype(jnp.int32), ((0, 0), (0, SPAD - SEQ)))
    idx2d = idxp.reshape(BATCH // ROWS_PER_STEP, WINDOW)
    x3 = _sc_gather(table_pad, idx2d)
    w1r = jnp.pad(W1.reshape(SEQ, EMB, HID), ((0, 0), (0, DPAD - EMB), (0, 0)))
    return _tc_mlp(x3, w1r, b1, W2, b2)


# speedup vs baseline: 6.6030x; 6.6030x over previous
import functools

import jax
import jax.numpy as jnp
from jax.experimental import pallas as pl
from jax.experimental.pallas import tpu as pltpu
from jax.experimental.pallas import tpu_sc as plsc

VOCAB = 20000
EMB = 50
SEQ = 42
BATCH = 16384
HID = 128
DPAD = 128
GATHER_WINDOW = 128
BLOCK_B = 512
NUM_CHUNKS = 4


def _sc_gather(table_pad, idx2d):
    n = idx2d.shape[1]
    mesh = plsc.VectorSubcoreMesh(core_axis_name="core", subcore_axis_name="subcore")

    @functools.partial(
        pl.kernel,
        out_type=jax.ShapeDtypeStruct((n, DPAD), table_pad.dtype),
        mesh=mesh,
    )
    def gather_kernel(table_hbm, i_hbm, o_hbm):
        def body(i_vmem, o_vmem):
            pltpu.sync_copy(table_hbm.at[i_vmem.at[0]], o_vmem)

        pltpu.emit_pipeline(
            body,
            grid=(n // GATHER_WINDOW,),
            in_specs=[pl.BlockSpec((1, GATHER_WINDOW), lambda i: (0, i))],
            out_specs=[pl.BlockSpec((GATHER_WINDOW, DPAD), lambda i: (i, 0))],
            core_axis_name=("core", "subcore"),
            dimension_semantics=(pltpu.PARALLEL,),
        )(i_hbm, o_hbm)

    return gather_kernel(table_pad, idx2d)


def _mlp_body(x_ref, w1_ref, b1_ref, w2_ref, b2_ref, o_ref):
    h = jnp.dot(x_ref[...], w1_ref[...], preferred_element_type=jnp.float32)
    h = jnp.maximum(h + b1_ref[...], 0.0)
    o = jnp.dot(h, w2_ref[...], preferred_element_type=jnp.float32) + b2_ref[...]
    o_ref[...] = jax.nn.sigmoid(o)


def _tc_mlp(x2, w1p, b1, w2, b2):
    rows = x2.shape[0]
    grid = (rows // BLOCK_B,)
    return pl.pallas_call(
        _mlp_body,
        grid=grid,
        in_specs=[
            pl.BlockSpec((BLOCK_B, SEQ * DPAD), lambda i: (i, 0)),
            pl.BlockSpec((SEQ * DPAD, HID), lambda i: (0, 0)),
            pl.BlockSpec((1, HID), lambda i: (0, 0)),
            pl.BlockSpec((HID, 1), lambda i: (0, 0)),
            pl.BlockSpec((1, 1), lambda i: (0, 0)),
        ],
        out_specs=pl.BlockSpec((BLOCK_B, 1), lambda i: (i, 0)),
        out_shape=jax.ShapeDtypeStruct((rows, 1), jnp.float32),
    )(x2, w1p, b1.reshape(1, HID), w2, b2.reshape(1, 1))


def kernel(indices, table, W1, b1, W2, b2):
    table_pad = jnp.pad(table, ((0, 0), (0, DPAD - EMB)))
    w1p = jnp.pad(
        W1.reshape(SEQ, EMB, HID), ((0, 0), (0, DPAD - EMB), (0, 0))
    ).reshape(SEQ * DPAD, HID)
    chunk = BATCH // NUM_CHUNKS
    idx_flat = indices.astype(jnp.int32).reshape(NUM_CHUNKS, 1, chunk * SEQ)
    outs = []
    for c in range(NUM_CHUNKS):
        x = _sc_gather(table_pad, idx_flat[c])
        x2 = x.reshape(chunk, SEQ * DPAD)
        outs.append(_tc_mlp(x2, w1p, b1, W2, b2))
    return jnp.concatenate(outs, axis=0)
